# baseline (speedup 1.0000x reference)
import numpy as np
import jax
import jax.numpy as jnp
from jax import lax
from jax.experimental import pallas as pl
from jax.experimental.pallas import tpu as pltpu

_VMEM_LIMIT = 64 * 1024 * 1024


def _resize_matrix(out_size, in_size):
    scale = in_size / out_size
    i = np.arange(out_size, dtype=np.float64)
    src = np.maximum((i + 0.5) * scale - 0.5, 0.0)
    i0 = np.minimum(np.floor(src).astype(np.int64), in_size - 1)
    i1 = np.minimum(i0 + 1, in_size - 1)
    lam1 = (src - i0).astype(np.float32)
    R = np.zeros((out_size, in_size), np.float32)
    R[np.arange(out_size), i0] += 1.0 - lam1
    R[np.arange(out_size), i1] += lam1
    return R


def _fused_kernel(x_ref, sk_ref, wup_ref, rh_ref, rwsT_ref, wt_ref,
                  beff_ref, bvec_ref, o_ref, s3_ref):
    Cin = x_ref.shape[1]
    Cs = sk_ref.shape[1]
    _, Hp, Wo, C2 = s3_ref.shape
    Ho = Hp - 2
    H, W = Ho // 2, Wo // 2
    Cout = o_ref.shape[1]
    Hs = rh_ref.shape[1]
    Ws = rwsT_ref.shape[1]
    bf = jnp.bfloat16

    xm = x_ref[0].astype(bf)
    up_flat = lax.dot_general(
        xm, wup_ref[...], (((0,), (0,)), ((), ())),
        preferred_element_type=jnp.float32)

    u00 = up_flat[:, 0 * Cout:1 * Cout].reshape(H, W, 1, Cout)
    u01 = up_flat[:, 1 * Cout:2 * Cout].reshape(H, W, 1, Cout)
    u10 = up_flat[:, 2 * Cout:3 * Cout].reshape(H, W, 1, Cout)
    u11 = up_flat[:, 3 * Cout:4 * Cout].reshape(H, W, 1, Cout)
    r0 = jnp.concatenate([u00, u01], axis=2).reshape(H, 1, Wo, Cout)
    r1 = jnp.concatenate([u10, u11], axis=2).reshape(H, 1, Wo, Cout)
    up_img = jnp.concatenate([r0, r1], axis=1).reshape(Ho, Wo, Cout)
    bcol = jnp.broadcast_to(bvec_ref[0, :Cout].reshape(1, 1, Cout),
                            (Ho, 1, Cout))
    up66 = jnp.concatenate([bcol, up_img, bcol], axis=1)

    sT = sk_ref[0].T.astype(bf)
    s3 = sT.reshape(Hs, Ws * Cs)
    th = jnp.dot(rh_ref[...], s3,
                 preferred_element_type=jnp.float32)
    th = th.reshape(Ho, Ws, Cs)
    th = jnp.transpose(th, (0, 2, 1)).astype(bf)
    th2 = th.reshape(Ho * Cs, Ws)

    brow = jnp.broadcast_to(bvec_ref[...].reshape(1, 1, C2), (1, Wo, C2))
    for kw in range(3):
        s3_ref[kw, 1:Ho + 1, :, 0:Cout] = up66[:, kw:kw + Wo, :].astype(bf)
        tw = jnp.dot(th2, rwsT_ref[kw],
                     preferred_element_type=jnp.float32)
        sk_kw = jnp.transpose(tw.reshape(Ho, Cs, Wo), (0, 2, 1))
        s3_ref[kw, 1:Ho + 1, :, Cout:C2] = sk_kw.astype(bf)
        s3_ref[kw, 0:1, :, :] = brow.astype(bf)
        s3_ref[kw, Hp - 1:Hp, :, :] = brow.astype(bf)

    y = jnp.zeros((Ho * Wo, Cout), jnp.float32)
    for kw in range(3):
        for kh in range(3):
            xt = s3_ref[kw, kh:kh + Ho].reshape(Ho * Wo, C2)
            y = y + jnp.dot(xt, wt_ref[3 * kw + kh],
                            preferred_element_type=jnp.float32)
    y = jnp.maximum(y + beff_ref[...], 0.0)
    o_ref[0] = y.T


def kernel(up_w, up_b, dw_w, dw_b, pw_w, pw_b,
           bn_gamma, bn_beta, bn_mean, bn_var, x, skip):
    N, Cin, H, W = x.shape
    _, Cs, Hs, Ws = skip.shape
    Cout = up_w.shape[1]
    C2 = 2 * Cout
    Ho, Wo = 2 * H, 2 * W

    wup = jnp.transpose(up_w, (0, 2, 3, 1)).reshape(Cin, 4 * Cout)
    wup = wup.astype(jnp.bfloat16)
    rh = jnp.asarray(_resize_matrix(Ho, Hs)).astype(jnp.bfloat16)
    rw_np = _resize_matrix(Wo, Ws)
    rwsT = np.zeros((3, Ws, Wo), np.float32)
    for kw in range(3):
        lo = max(0, 1 - kw)
        hi = min(Wo, Wo + 1 - kw)
        rwsT[kw, :, lo:hi] = rw_np[lo + kw - 1:hi + kw - 1].T
    rwsT = jnp.asarray(rwsT).astype(jnp.bfloat16)
    wdw = jnp.transpose(dw_w[:, 0, :, :], (1, 2, 0))
    inv = bn_gamma / jnp.sqrt(bn_var + 1e-5)
    wpw = jnp.transpose(pw_w[:, :, 0, 0], (1, 0)) * inv[None, :]
    sdw = jnp.sum(wdw, axis=(0, 1))
    beff = (pw_b * inv + bn_beta - bn_mean * inv
            + dw_b @ wpw + (up_b * sdw[:Cout]) @ wpw[:Cout]).reshape(1, Cout)
    bvec = jnp.concatenate([-up_b, jnp.zeros((Cout,), jnp.float32)])
    bvec = bvec.reshape(1, C2)
    wdw_t = jnp.transpose(wdw, (1, 0, 2)).reshape(9, C2, 1)
    wt = (wdw_t * wpw.reshape(1, C2, Cout)).astype(jnp.bfloat16)

    x_flat = x.reshape(N, Cin, H * W)
    sk_flat = skip.reshape(N, Cs, Hs * Ws)

    out = pl.pallas_call(
        _fused_kernel,
        out_shape=jax.ShapeDtypeStruct((N, Cout, Ho * Wo), jnp.float32),
        grid=(N,),
        in_specs=[
            pl.BlockSpec((1, Cin, H * W), lambda n: (n, 0, 0)),
            pl.BlockSpec((1, Cs, Hs * Ws), lambda n: (n, 0, 0)),
            pl.BlockSpec((Cin, 4 * Cout), lambda n: (0, 0)),
            pl.BlockSpec((Ho, Hs), lambda n: (0, 0)),
            pl.BlockSpec((3, Ws, Wo), lambda n: (0, 0, 0)),
            pl.BlockSpec((9, C2, Cout), lambda n: (0, 0, 0)),
            pl.BlockSpec((1, Cout), lambda n: (0, 0)),
            pl.BlockSpec((1, C2), lambda n: (0, 0)),
        ],
        out_specs=pl.BlockSpec((1, Cout, Ho * Wo), lambda n: (n, 0, 0)),
        scratch_shapes=[pltpu.VMEM((3, Ho + 2, Wo, C2), jnp.bfloat16)],
        compiler_params=pltpu.CompilerParams(
            dimension_semantics=("parallel",),
            vmem_limit_bytes=_VMEM_LIMIT),
    )(x_flat, sk_flat, wup, rh, rwsT, wt, beff, bvec)

    return out.reshape(N, Cout, Ho, Wo)


# speedup vs baseline: 1.3842x; 1.3842x over previous
import numpy as np
import jax
import jax.numpy as jnp
from jax import lax
from jax.experimental import pallas as pl
from jax.experimental.pallas import tpu as pltpu

_VMEM_LIMIT = 64 * 1024 * 1024


def _resize_matrix(out_size, in_size):
    scale = in_size / out_size
    i = np.arange(out_size, dtype=np.float64)
    src = np.maximum((i + 0.5) * scale - 0.5, 0.0)
    i0 = np.minimum(np.floor(src).astype(np.int64), in_size - 1)
    i1 = np.minimum(i0 + 1, in_size - 1)
    lam1 = (src - i0).astype(np.float32)
    R = np.zeros((out_size, in_size), np.float32)
    R[np.arange(out_size), i0] += 1.0 - lam1
    R[np.arange(out_size), i1] += lam1
    return R


def _fused_kernel(x_ref, sk_ref, wup_ref, rh_ref, rwsT_ref, wdw_ref,
                  wpw_ref, beff_ref, bvec_ref, o_ref):
    Cin = x_ref.shape[1]
    Cs = sk_ref.shape[1]
    Cout = o_ref.shape[1]
    Hs = rh_ref.shape[1]
    Ws = rwsT_ref.shape[1]
    Wo = rwsT_ref.shape[2]
    Ho = rh_ref.shape[0]
    H, W = Ho // 2, Wo // 2
    C2 = bvec_ref.shape[1]
    bf = jnp.bfloat16

    xm = x_ref[0].astype(bf)
    up_flat = lax.dot_general(
        xm, wup_ref[...], (((0,), (0,)), ((), ())),
        preferred_element_type=jnp.float32)

    u00 = up_flat[:, 0 * Cout:1 * Cout].reshape(H, W, 1, Cout)
    u01 = up_flat[:, 1 * Cout:2 * Cout].reshape(H, W, 1, Cout)
    u10 = up_flat[:, 2 * Cout:3 * Cout].reshape(H, W, 1, Cout)
    u11 = up_flat[:, 3 * Cout:4 * Cout].reshape(H, W, 1, Cout)
    r0 = jnp.concatenate([u00, u01], axis=2).reshape(H, 1, Wo, Cout)
    r1 = jnp.concatenate([u10, u11], axis=2).reshape(H, 1, Wo, Cout)
    up_img = jnp.concatenate([r0, r1], axis=1).reshape(Ho, Wo, Cout)
    bcol = jnp.broadcast_to(bvec_ref[0, :Cout].reshape(1, 1, Cout),
                            (Ho, 1, Cout))
    up66 = jnp.concatenate([bcol, up_img, bcol], axis=1)

    sT = sk_ref[0].T.astype(bf)
    s3 = sT.reshape(Hs, Ws * Cs)
    th = jnp.dot(rh_ref[...], s3,
                 preferred_element_type=jnp.float32)
    th = th.reshape(Ho, Ws, Cs)
    th = jnp.transpose(th, (0, 2, 1)).astype(bf)
    th2 = th.reshape(Ho * Cs, Ws)

    wdw = wdw_ref[...]
    brow_u = jnp.broadcast_to(bvec_ref[0, :Cout].reshape(1, 1, Cout),
                              (1, Wo, Cout)).astype(bf)
    zrow = jnp.zeros((1, Wo, Cs), bf)
    acc_u = jnp.zeros((Ho, Wo, Cout), bf)
    acc_s = jnp.zeros((Ho, Wo, Cs), bf)
    for kw in range(3):
        pu = jnp.concatenate(
            [brow_u, up66[:, kw:kw + Wo, :].astype(bf), brow_u], axis=0)
        tw = jnp.dot(th2, rwsT_ref[kw],
                     preferred_element_type=jnp.float32)
        sk_kw = jnp.transpose(tw.reshape(Ho, Cs, Wo), (0, 2, 1))
        ps = jnp.concatenate([zrow, sk_kw.astype(bf), zrow], axis=0)
        for kh in range(3):
            acc_u = acc_u + pu[kh:kh + Ho] * wdw[kh, kw, :Cout]
            acc_s = acc_s + ps[kh:kh + Ho] * wdw[kh, kw, Cout:]

    y = (jnp.dot(acc_u.reshape(Ho * Wo, Cout), wpw_ref[0:Cout],
                 preferred_element_type=jnp.float32)
         + jnp.dot(acc_s.reshape(Ho * Wo, Cs), wpw_ref[Cout:C2],
                   preferred_element_type=jnp.float32))
    y = jnp.maximum(y + beff_ref[...], 0.0)
    o_ref[0] = y.T


def kernel(up_w, up_b, dw_w, dw_b, pw_w, pw_b,
           bn_gamma, bn_beta, bn_mean, bn_var, x, skip):
    N, Cin, H, W = x.shape
    _, Cs, Hs, Ws = skip.shape
    Cout = up_w.shape[1]
    C2 = 2 * Cout
    Ho, Wo = 2 * H, 2 * W

    wup = jnp.transpose(up_w, (0, 2, 3, 1)).reshape(Cin, 4 * Cout)
    wup = wup.astype(jnp.bfloat16)
    rh = jnp.asarray(_resize_matrix(Ho, Hs)).astype(jnp.bfloat16)
    rw_np = _resize_matrix(Wo, Ws)
    rwsT = np.zeros((3, Ws, Wo), np.float32)
    for kw in range(3):
        lo = max(0, 1 - kw)
        hi = min(Wo, Wo + 1 - kw)
        rwsT[kw, :, lo:hi] = rw_np[lo + kw - 1:hi + kw - 1].T
    rwsT = jnp.asarray(rwsT).astype(jnp.bfloat16)
    wdw = jnp.transpose(dw_w[:, 0, :, :], (1, 2, 0))
    inv = bn_gamma / jnp.sqrt(bn_var + 1e-5)
    wpw = jnp.transpose(pw_w[:, :, 0, 0], (1, 0)) * inv[None, :]
    sdw = jnp.sum(wdw, axis=(0, 1))
    beff = (pw_b * inv + bn_beta - bn_mean * inv
            + dw_b @ wpw + (up_b * sdw[:Cout]) @ wpw[:Cout]).reshape(1, Cout)
    bvec = jnp.concatenate([-up_b, jnp.zeros((Cout,), jnp.float32)])
    bvec = bvec.reshape(1, C2)
    wdw = wdw.astype(jnp.bfloat16)
    wpw = wpw.astype(jnp.bfloat16)

    x_flat = x.reshape(N, Cin, H * W)
    sk_flat = skip.reshape(N, Cs, Hs * Ws)

    out = pl.pallas_call(
        _fused_kernel,
        out_shape=jax.ShapeDtypeStruct((N, Cout, Ho * Wo), jnp.float32),
        grid=(N,),
        in_specs=[
            pl.BlockSpec((1, Cin, H * W), lambda n: (n, 0, 0)),
            pl.BlockSpec((1, Cs, Hs * Ws), lambda n: (n, 0, 0)),
            pl.BlockSpec((Cin, 4 * Cout), lambda n: (0, 0)),
            pl.BlockSpec((Ho, Hs), lambda n: (0, 0)),
            pl.BlockSpec((3, Ws, Wo), lambda n: (0, 0, 0)),
            pl.BlockSpec((3, 3, C2), lambda n: (0, 0, 0)),
            pl.BlockSpec((C2, Cout), lambda n: (0, 0)),
            pl.BlockSpec((1, Cout), lambda n: (0, 0)),
            pl.BlockSpec((1, C2), lambda n: (0, 0)),
        ],
        out_specs=pl.BlockSpec((1, Cout, Ho * Wo), lambda n: (n, 0, 0)),
        compiler_params=pltpu.CompilerParams(
            dimension_semantics=("parallel",),
            vmem_limit_bytes=_VMEM_LIMIT),
    )(x_flat, sk_flat, wup, rh, rwsT, wdw, wpw, beff, bvec)

    return out.reshape(N, Cout, Ho, Wo)
